# baseline (device time: 60532 ns/iter reference)
import jax
import jax.numpy as jnp
from jax import lax
from jax.experimental import pallas as pl
from jax.experimental.pallas import tpu as pltpu


def kernel(x, W):
    t, d = x.shape
    _, v = W.shape

    def body(x_ref, w_ref, out_ref, mine_ref, theirs_ref, send_sem, recv_sem):
        my_x = lax.axis_index("x")
        my_y = lax.axis_index("y")
        partner = (1 - my_x, my_y)

        barrier_sem = pltpu.get_barrier_semaphore()
        pl.semaphore_signal(
            barrier_sem, inc=1,
            device_id=partner, device_id_type=pl.DeviceIdType.MESH,
        )
        pl.semaphore_wait(barrier_sem, 1)

        mine_ref[:, :] = jnp.dot(
            x_ref[:, :], w_ref[:, :], preferred_element_type=jnp.float32
        )

        rdma = pltpu.make_async_remote_copy(
            src_ref=mine_ref,
            dst_ref=theirs_ref,
            send_sem=send_sem,
            recv_sem=recv_sem,
            device_id=partner,
            device_id_type=pl.DeviceIdType.MESH,
        )
        rdma.start()
        rdma.wait()

        @pl.when(my_x == 0)
        def _():
            out_ref[:, 0:v] = mine_ref[:, :]
            out_ref[:, v : 2 * v] = theirs_ref[:, :]

        @pl.when(my_x == 1)
        def _():
            out_ref[:, 0:v] = theirs_ref[:, :]
            out_ref[:, v : 2 * v] = mine_ref[:, :]

        logits = out_ref[:, :]
        m = jnp.max(logits, axis=-1, keepdims=True)
        e = jnp.exp(logits - m)
        out_ref[:, :] = e / jnp.sum(e, axis=-1, keepdims=True)

    return pl.pallas_call(
        body,
        out_shape=jax.ShapeDtypeStruct((t, 2 * v), jnp.float32),
        in_specs=[
            pl.BlockSpec(memory_space=pltpu.VMEM),
            pl.BlockSpec(memory_space=pltpu.VMEM),
        ],
        out_specs=pl.BlockSpec(memory_space=pltpu.VMEM),
        scratch_shapes=[
            pltpu.VMEM((t, v), jnp.float32),
            pltpu.VMEM((t, v), jnp.float32),
            pltpu.SemaphoreType.DMA,
            pltpu.SemaphoreType.DMA,
        ],
        compiler_params=pltpu.CompilerParams(collective_id=0),
    )(x, W)


# device time: 57741 ns/iter; 1.0483x vs baseline; 1.0483x over previous
import jax
import jax.numpy as jnp
from jax import lax
from jax.experimental import pallas as pl
from jax.experimental.pallas import tpu as pltpu

C = 8


def kernel(x, W):
    t, d = x.shape
    _, v = W.shape
    tc = t // C

    def body(x_ref, w_ref, out_ref, mine_ref, theirs_ref, send_sems, recv_sems):
        my_x = lax.axis_index("x")
        my_y = lax.axis_index("y")
        partner = (1 - my_x, my_y)

        barrier_sem = pltpu.get_barrier_semaphore()
        pl.semaphore_signal(
            barrier_sem, inc=1,
            device_id=partner, device_id_type=pl.DeviceIdType.MESH,
        )
        pl.semaphore_wait(barrier_sem, 1)

        rdmas = []
        for c in range(C):
            rows = pl.ds(c * tc, tc)
            mine_ref[rows, :] = jnp.dot(
                x_ref[rows, :], w_ref[:, :], preferred_element_type=jnp.float32
            )
            rdma = pltpu.make_async_remote_copy(
                src_ref=mine_ref.at[rows, :],
                dst_ref=theirs_ref.at[rows, :],
                send_sem=send_sems.at[c],
                recv_sem=recv_sems.at[c],
                device_id=partner,
                device_id_type=pl.DeviceIdType.MESH,
            )
            rdma.start()
            rdmas.append(rdma)

        my_off = my_x * v
        their_off = (1 - my_x) * v
        for c in range(C):
            rows = pl.ds(c * tc, tc)
            rdmas[c].wait_recv()
            mi = mine_ref[rows, :]
            th = theirs_ref[rows, :]
            m = jnp.maximum(
                jnp.max(mi, axis=-1, keepdims=True),
                jnp.max(th, axis=-1, keepdims=True),
            )
            em = jnp.exp(mi - m)
            et = jnp.exp(th - m)
            inv = 1.0 / (
                jnp.sum(em, axis=-1, keepdims=True)
                + jnp.sum(et, axis=-1, keepdims=True)
            )
            out_ref[rows, pl.ds(my_off, v)] = em * inv
            out_ref[rows, pl.ds(their_off, v)] = et * inv

        for c in range(C):
            rdmas[c].wait_send()

    return pl.pallas_call(
        body,
        out_shape=jax.ShapeDtypeStruct((t, 2 * v), jnp.float32),
        in_specs=[
            pl.BlockSpec(memory_space=pltpu.VMEM),
            pl.BlockSpec(memory_space=pltpu.VMEM),
        ],
        out_specs=pl.BlockSpec(memory_space=pltpu.VMEM),
        scratch_shapes=[
            pltpu.VMEM((t, v), jnp.float32),
            pltpu.VMEM((t, v), jnp.float32),
            pltpu.SemaphoreType.DMA((C,)),
            pltpu.SemaphoreType.DMA((C,)),
        ],
        compiler_params=pltpu.CompilerParams(collective_id=0),
    )(x, W)
